# baseline (speedup 1.0000x reference)
import functools

import jax
import jax.numpy as jnp
from jax import lax
from jax.experimental import pallas as pl
from jax.experimental.pallas import tpu as pltpu
from jax.experimental.pallas import tpu_sc as plsc

_N_FIELDS = 26
_VOCAB = 100000
_EMB = 16
_NUM_CORES = 2
_NUM_SUBCORES = 16
_LANES = 16


@functools.partial(jax.jit, static_argnums=(2,))
def _embedding_gather(x, tables, bchunk):
    batch = x.shape[0]
    n_workers = _NUM_CORES * _NUM_SUBCORES
    b_per_w = batch // n_workers
    chunk = bchunk * _N_FIELDS
    n_chunks = b_per_w // bchunk
    qgroups = bchunk // _LANES
    mesh = plsc.VectorSubcoreMesh(core_axis_name="c", subcore_axis_name="s")

    def body(x_hbm, tab_hbm, out_hbm, xv, idx_t, rows_v, gsem, wsem):
        wid = lax.axis_index("s") * _NUM_CORES + lax.axis_index("c")
        lanes = lax.iota(jnp.int32, _LANES)

        pltpu.sync_copy(x_hbm.at[pl.ds(wid * b_per_w, b_per_w), :], xv)

        def do_chunk(g, carry):
            for f in range(_N_FIELDS):
                fvec = lanes * 0 + f
                for q in range(qgroups):
                    bvec = g * bchunk + q * _LANES + lanes
                    vals = plsc.load_gather(xv, [bvec, fvec])
                    idx_t[pl.ds(f * bchunk + q * _LANES, _LANES)] = (
                        vals + f * _VOCAB)

            pltpu.async_copy(tab_hbm.at[0].at[idx_t], rows_v, gsem).wait()

            b0 = wid * b_per_w + g * bchunk
            copies = []
            for f in range(_N_FIELDS):
                copies.append(pltpu.async_copy(
                    rows_v.at[pl.ds(f * bchunk, bchunk), :],
                    out_hbm.at[pl.ds(b0, bchunk), pl.ds(f * _EMB, _EMB)],
                    wsem))
            for c in copies:
                c.wait()
            return carry

        lax.fori_loop(0, n_chunks, do_chunk, 0)

    return pl.kernel(
        body,
        out_type=jax.ShapeDtypeStruct((batch, _N_FIELDS * _EMB), jnp.float32),
        mesh=mesh,
        scratch_types=[
            pltpu.VMEM((b_per_w, _N_FIELDS), jnp.int32),
            pltpu.VMEM((chunk,), jnp.int32),
            pltpu.VMEM((chunk, _EMB), jnp.float32),
            pltpu.SemaphoreType.DMA,
            pltpu.SemaphoreType.DMA,
        ],
        compiler_params=pltpu.CompilerParams(use_tc_tiling_on_sc=False,
                                             needs_layout_passes=False,
                                             disable_bounds_checks=True),
    )(x, tables)


def kernel(x, tables):
    return _embedding_gather(x, tables, 64)


# speedup vs baseline: 2.3707x; 2.3707x over previous
import functools

import jax
import jax.numpy as jnp
from jax import lax
from jax.experimental import pallas as pl
from jax.experimental.pallas import tpu as pltpu
from jax.experimental.pallas import tpu_sc as plsc

_N_FIELDS = 26
_VOCAB = 100000
_EMB = 16
_NUM_CORES = 2
_NUM_SUBCORES = 16
_LANES = 16
_BCHUNK = 2048
_UNROLL = 4


@functools.partial(jax.jit, static_argnums=())
def _embedding_gather_t(x_t, tab_t):
    n_fields, batch = x_t.shape
    n_cols = n_fields * _EMB
    n_workers = _NUM_CORES * _NUM_SUBCORES
    cols_per_w = n_cols // n_workers
    n_chunks = batch // _BCHUNK
    groups = _BCHUNK // (_LANES * _UNROLL)
    mesh = plsc.VectorSubcoreMesh(core_axis_name="c", subcore_axis_name="s")

    def body(x_hbm, tab_hbm, out_hbm, col_v, idx_v, out_v):
        wid = lax.axis_index("s") * _NUM_CORES + lax.axis_index("c")

        def do_col(k, carry):
            j = wid * cols_per_w + k
            f = j // _EMB
            c = j % _EMB
            pltpu.sync_copy(tab_hbm.at[f, c, :], col_v)

            def do_chunk(h, carry2):
                b0 = h * _BCHUNK
                pltpu.sync_copy(x_hbm.at[f, pl.ds(b0, _BCHUNK)], idx_v)

                def gather16(q, carry3):
                    for u in range(_UNROLL):
                        o = (q * _UNROLL + u) * _LANES
                        iv = idx_v[pl.ds(o, _LANES)]
                        out_v[pl.ds(o, _LANES)] = plsc.load_gather(
                            col_v, [iv])
                    return carry3

                lax.fori_loop(0, groups, gather16, 0)
                pltpu.sync_copy(out_v, out_hbm.at[j, pl.ds(b0, _BCHUNK)])
                return carry2

            lax.fori_loop(0, n_chunks, do_chunk, 0)
            return carry

        lax.fori_loop(0, cols_per_w, do_col, 0)

    return pl.kernel(
        body,
        out_type=jax.ShapeDtypeStruct((n_cols, batch), jnp.float32),
        mesh=mesh,
        scratch_types=[
            pltpu.VMEM((_VOCAB,), jnp.float32),
            pltpu.VMEM((_BCHUNK,), jnp.int32),
            pltpu.VMEM((_BCHUNK,), jnp.float32),
        ],
        compiler_params=pltpu.CompilerParams(use_tc_tiling_on_sc=False,
                                             needs_layout_passes=False),
    )(x_t, tab_t)


def kernel(x, tables):
    x_t = x.T
    tab_t = jnp.transpose(tables, (0, 2, 1))
    out_t = _embedding_gather_t(x_t, tab_t)
    return out_t.T


# speedup vs baseline: 5.1005x; 2.1514x over previous
import functools

import jax
import jax.numpy as jnp
from jax import lax
from jax.experimental import pallas as pl
from jax.experimental.pallas import tpu as pltpu
from jax.experimental.pallas import tpu_sc as plsc

_N_FIELDS = 26
_VOCAB = 100000
_EMB = 16
_NUM_CORES = 2
_NUM_SUBCORES = 16
_LANES = 16
_BCHUNK = 2048
_UNROLL = 4


@functools.partial(jax.jit, static_argnums=())
def _embedding_gather_t(x_t, tab_t):
    n_fields, batch = x_t.shape
    n_cols = n_fields * _EMB
    n_workers = _NUM_CORES * _NUM_SUBCORES
    cols_per_w = n_cols // n_workers
    n_chunks = batch // _BCHUNK
    groups = _BCHUNK // (_LANES * _UNROLL)
    mesh = plsc.VectorSubcoreMesh(core_axis_name="c", subcore_axis_name="s")

    def body(x_hbm, tab_hbm, out_hbm, col_v, idx_v, out_v):
        wid = lax.axis_index("s") * _NUM_CORES + lax.axis_index("c")

        def do_col(k, carry):
            j = wid * cols_per_w + k
            f = j // _EMB
            c = j % _EMB
            pltpu.sync_copy(tab_hbm.at[f, c, :], col_v)

            def do_chunk(h, carry2):
                b0 = h * _BCHUNK
                pltpu.sync_copy(x_hbm.at[f, pl.ds(b0, _BCHUNK)], idx_v)

                def gather16(q, carry3):
                    for u in range(_UNROLL):
                        o = (q * _UNROLL + u) * _LANES
                        iv = idx_v[pl.ds(o, _LANES)]
                        out_v[pl.ds(o, _LANES)] = plsc.load_gather(
                            col_v, [iv])
                    return carry3

                lax.fori_loop(0, groups, gather16, 0)
                pltpu.sync_copy(out_v, out_hbm.at[j, pl.ds(b0, _BCHUNK)])
                return carry2

            lax.fori_loop(0, n_chunks, do_chunk, 0)
            return carry

        lax.fori_loop(0, cols_per_w, do_col, 0)

    return pl.kernel(
        body,
        out_type=jax.ShapeDtypeStruct((n_cols, batch), jnp.float32),
        mesh=mesh,
        scratch_types=[
            pltpu.VMEM((_VOCAB,), jnp.float32),
            pltpu.VMEM((_BCHUNK,), jnp.int32),
            pltpu.VMEM((_BCHUNK,), jnp.float32),
        ],
        compiler_params=pltpu.CompilerParams(use_tc_tiling_on_sc=True,
                                             needs_layout_passes=False),
    )(x_t, tab_t)


def kernel(x, tables):
    x_t = x.T
    tab_t = jnp.transpose(tables, (0, 2, 1))
    out_t = _embedding_gather_t(x_t, tab_t)
    return out_t.T
